# Initial kernel scaffold; baseline (speedup 1.0000x reference)
import jax
import jax.numpy as jnp
from jax.experimental import pallas as pl


def kernel(inputs, codebook):
    raise NotImplementedError("write your pallas kernel here")


# speedup vs baseline: 5.3299x; 5.3299x over previous
import functools

import jax
import jax.numpy as jnp
from jax.experimental import pallas as pl

_TBLK = 256


def _vq_block(x_ref, cbt_ref, cb_ref, q_ref, loss_ref, idx_ref):
    x = x_ref[:]
    cbt = cbt_ref[:]
    k = cbt.shape[1]

    cnorm = jnp.sum(cbt * cbt, axis=0, keepdims=True)
    dots = jax.lax.dot_general(
        x, cbt, (((1,), (0,)), ((), ())),
        preferred_element_type=jnp.float32,
        precision=jax.lax.Precision.HIGHEST,
    )
    scores = cnorm - 2.0 * dots

    mins = jnp.min(scores, axis=1, keepdims=True)
    iota = jax.lax.broadcasted_iota(jnp.int32, scores.shape, 1)
    loc = jnp.min(jnp.where(scores == mins, iota, k),
                  axis=1, keepdims=True)

    one_hot = (iota == loc).astype(jnp.float32)
    q = jax.lax.dot_general(
        one_hot, cb_ref[:], (((1,), (0,)), ((), ())),
        preferred_element_type=jnp.float32,
        precision=jax.lax.Precision.HIGHEST,
    )

    diff = q - x
    loss_ref[:] = diff * diff
    q_ref[:] = x + diff
    idx_ref[:] = loc


@functools.partial(jax.jit, static_argnames=())
def _vq(flat_x, codebook_t, codebook):
    n, d = flat_x.shape
    k = codebook.shape[0]
    nblk = n // _TBLK
    q, loss, idx = pl.pallas_call(
        _vq_block,
        grid=(nblk,),
        in_specs=[
            pl.BlockSpec((_TBLK, d), lambda i: (i, 0)),
            pl.BlockSpec((d, k), lambda i: (0, 0)),
            pl.BlockSpec((k, d), lambda i: (0, 0)),
        ],
        out_specs=[
            pl.BlockSpec((_TBLK, d), lambda i: (i, 0)),
            pl.BlockSpec((_TBLK, d), lambda i: (i, 0)),
            pl.BlockSpec((_TBLK, 1), lambda i: (i, 0)),
        ],
        out_shape=[
            jax.ShapeDtypeStruct((n, d), jnp.float32),
            jax.ShapeDtypeStruct((n, d), jnp.float32),
            jax.ShapeDtypeStruct((n, 1), jnp.int32),
        ],
    )(flat_x, codebook_t, codebook)
    return q, loss, idx


def kernel(inputs, codebook):
    b, t, d = inputs.shape
    flat = inputs.reshape(b * t, d)
    q, loss, idx = _vq(flat, codebook.T, codebook)
    quantized = q.reshape(1, b, t, d)
    quantization_loss = loss.reshape(1, b, t, d)
    nn_idx = idx.reshape(1, b, t)
    codebook_out = jax.lax.stop_gradient(codebook[None])
    return (quantized, quantization_loss, nn_idx, codebook_out)


# speedup vs baseline: 10.3921x; 1.9498x over previous
import functools

import jax
import jax.numpy as jnp
from jax.experimental import pallas as pl

_TBLK = 512


def _vq_block(x_ref, cbt_ref, cb_ref, q_ref, loss_ref, idx_ref):
    x = x_ref[:]
    cbt = cbt_ref[:]
    k = cbt.shape[1]

    cnorm = jnp.sum(cbt * cbt, axis=0, keepdims=True)
    dots = jax.lax.dot_general(
        x, cbt, (((1,), (0,)), ((), ())),
        preferred_element_type=jnp.float32,
        precision=jax.lax.Precision.HIGHEST,
    )
    scores = cnorm - 2.0 * dots

    mins = jnp.min(scores, axis=1, keepdims=True)
    iota = jax.lax.broadcasted_iota(jnp.int32, scores.shape, 1)
    loc = jnp.min(jnp.where(scores == mins, iota, k),
                  axis=1, keepdims=True)

    one_hot = (iota == loc).astype(jnp.float32)
    q = jax.lax.dot_general(
        one_hot, cb_ref[:], (((1,), (0,)), ((), ())),
        preferred_element_type=jnp.float32,
        precision=jax.lax.Precision.DEFAULT,
    )

    diff = q - x
    loss_ref[:] = diff * diff
    q_ref[:] = x + diff
    idx_ref[:] = loc


@functools.partial(jax.jit, static_argnames=())
def _vq(flat_x, codebook_t, codebook):
    n, d = flat_x.shape
    k = codebook.shape[0]
    nblk = n // _TBLK
    q, loss, idx = pl.pallas_call(
        _vq_block,
        grid=(nblk,),
        in_specs=[
            pl.BlockSpec((_TBLK, d), lambda i: (i, 0)),
            pl.BlockSpec((d, k), lambda i: (0, 0)),
            pl.BlockSpec((k, d), lambda i: (0, 0)),
        ],
        out_specs=[
            pl.BlockSpec((_TBLK, d), lambda i: (i, 0)),
            pl.BlockSpec((_TBLK, d), lambda i: (i, 0)),
            pl.BlockSpec((_TBLK, 1), lambda i: (i, 0)),
        ],
        out_shape=[
            jax.ShapeDtypeStruct((n, d), jnp.float32),
            jax.ShapeDtypeStruct((n, d), jnp.float32),
            jax.ShapeDtypeStruct((n, 1), jnp.int32),
        ],
    )(flat_x, codebook_t, codebook)
    return q, loss, idx


def kernel(inputs, codebook):
    b, t, d = inputs.shape
    flat = inputs.reshape(b * t, d)
    q, loss, idx = _vq(flat, codebook.T, codebook)
    quantized = q.reshape(1, b, t, d)
    quantization_loss = loss.reshape(1, b, t, d)
    nn_idx = idx.reshape(1, b, t)
    codebook_out = jax.lax.stop_gradient(codebook[None])
    return (quantized, quantization_loss, nn_idx, codebook_out)


# speedup vs baseline: 11.0079x; 1.0593x over previous
import functools

import jax
import jax.numpy as jnp
from jax.experimental import pallas as pl

_TBLK = 2304


def _vq_block(x_ref, cbt_ref, cb_ref, q_ref, loss_ref, idx_ref):
    x = x_ref[:]
    cbt = cbt_ref[:]
    k = cbt.shape[1]

    cnorm = jnp.sum(cbt * cbt, axis=0, keepdims=True)
    dots = jax.lax.dot_general(
        x, cbt, (((1,), (0,)), ((), ())),
        preferred_element_type=jnp.float32,
        precision=jax.lax.Precision.HIGHEST,
    )
    scores = cnorm - 2.0 * dots

    mins = jnp.min(scores, axis=1, keepdims=True)
    iota = jax.lax.broadcasted_iota(jnp.int32, scores.shape, 1)
    loc = jnp.min(jnp.where(scores == mins, iota, k),
                  axis=1, keepdims=True)

    one_hot = (iota == loc).astype(jnp.float32)
    q = jax.lax.dot_general(
        one_hot, cb_ref[:], (((1,), (0,)), ((), ())),
        preferred_element_type=jnp.float32,
        precision=jax.lax.Precision.DEFAULT,
    )

    diff = q - x
    loss_ref[:] = diff * diff
    q_ref[:] = x + diff
    idx_ref[:] = loc


@functools.partial(jax.jit, static_argnames=())
def _vq(flat_x, codebook_t, codebook):
    n, d = flat_x.shape
    k = codebook.shape[0]
    nblk = n // _TBLK
    q, loss, idx = pl.pallas_call(
        _vq_block,
        grid=(nblk,),
        in_specs=[
            pl.BlockSpec((_TBLK, d), lambda i: (i, 0)),
            pl.BlockSpec((d, k), lambda i: (0, 0)),
            pl.BlockSpec((k, d), lambda i: (0, 0)),
        ],
        out_specs=[
            pl.BlockSpec((_TBLK, d), lambda i: (i, 0)),
            pl.BlockSpec((_TBLK, d), lambda i: (i, 0)),
            pl.BlockSpec((_TBLK, 1), lambda i: (i, 0)),
        ],
        out_shape=[
            jax.ShapeDtypeStruct((n, d), jnp.float32),
            jax.ShapeDtypeStruct((n, d), jnp.float32),
            jax.ShapeDtypeStruct((n, 1), jnp.int32),
        ],
    )(flat_x, codebook_t, codebook)
    return q, loss, idx


def kernel(inputs, codebook):
    b, t, d = inputs.shape
    flat = inputs.reshape(b * t, d)
    q, loss, idx = _vq(flat, codebook.T, codebook)
    quantized = q.reshape(1, b, t, d)
    quantization_loss = loss.reshape(1, b, t, d)
    nn_idx = idx.reshape(1, b, t)
    codebook_out = jax.lax.stop_gradient(codebook[None])
    return (quantized, quantization_loss, nn_idx, codebook_out)


# speedup vs baseline: 11.0632x; 1.0050x over previous
import functools

import jax
import jax.numpy as jnp
from jax.experimental import pallas as pl

_TBLK = 2304


def _vq_block(x_ref, cbt_ref, cb_ref, q_ref, loss_ref, idx_ref):
    x = x_ref[:]
    cbt = cbt_ref[:]
    k = cbt.shape[1]

    cnorm = jnp.sum(cbt * cbt, axis=0, keepdims=True)
    dots = jax.lax.dot_general(
        x, cbt, (((1,), (0,)), ((), ())),
        preferred_element_type=jnp.float32,
        precision=jax.lax.Precision.HIGHEST,
    )
    scores = cnorm - 2.0 * dots

    mins = jnp.min(scores, axis=1, keepdims=True)
    iota = jax.lax.broadcasted_iota(jnp.int32, scores.shape, 1)
    loc = jnp.min(jnp.where(scores == mins, iota, k),
                  axis=1, keepdims=True)

    one_hot = (iota == loc).astype(jnp.float32)
    q = jax.lax.dot_general(
        one_hot, cb_ref[:], (((1,), (0,)), ((), ())),
        preferred_element_type=jnp.float32,
        precision=jax.lax.Precision.DEFAULT,
    )

    diff = q - x
    loss_ref[:] = diff * diff
    q_ref[:] = x + diff
    idx_ref[:] = loc.reshape(1, x.shape[0])


@functools.partial(jax.jit, static_argnames=())
def _vq(flat_x, codebook_t, codebook):
    n, d = flat_x.shape
    k = codebook.shape[0]
    nblk = n // _TBLK
    q, loss, idx = pl.pallas_call(
        _vq_block,
        grid=(nblk,),
        in_specs=[
            pl.BlockSpec((_TBLK, d), lambda i: (i, 0)),
            pl.BlockSpec((d, k), lambda i: (0, 0)),
            pl.BlockSpec((k, d), lambda i: (0, 0)),
        ],
        out_specs=[
            pl.BlockSpec((_TBLK, d), lambda i: (i, 0)),
            pl.BlockSpec((_TBLK, d), lambda i: (i, 0)),
            pl.BlockSpec((1, _TBLK), lambda i: (0, i)),
        ],
        out_shape=[
            jax.ShapeDtypeStruct((n, d), jnp.float32),
            jax.ShapeDtypeStruct((n, d), jnp.float32),
            jax.ShapeDtypeStruct((1, n), jnp.int32),
        ],
    )(flat_x, codebook_t, codebook)
    return q, loss, idx


def kernel(inputs, codebook):
    b, t, d = inputs.shape
    flat = inputs.reshape(b * t, d)
    q, loss, idx = _vq(flat, codebook.T, codebook)
    quantized = q.reshape(1, b, t, d)
    quantization_loss = loss.reshape(1, b, t, d)
    nn_idx = idx.reshape(1, b, t)
    codebook_out = jax.lax.stop_gradient(codebook[None])
    return (quantized, quantization_loss, nn_idx, codebook_out)
